# Initial kernel scaffold; baseline (speedup 1.0000x reference)
import jax
import jax.numpy as jnp
from jax.experimental import pallas as pl


def kernel(positions, adj, params):
    raise NotImplementedError("write your pallas kernel here")


# speedup vs baseline: 1.1402x; 1.1402x over previous
import jax
import jax.numpy as jnp
from jax.experimental import pallas as pl
from jax.experimental.pallas import tpu as pltpu


def _elu(v):
    return jnp.where(v > 0, v, jnp.exp(jnp.minimum(v, 0.0)) - 1.0)


def _encoder_body(nlayers):
    def body(*refs):
        adj_ref, pos_ref = refs[0], refs[1]
        out_ref = refs[-1]
        wb = refs[2:-1]
        adj = adj_ref[0]
        x = pos_ref[0]
        for i in range(nlayers - 1):
            w = wb[2 * i][...]
            b = wb[2 * i + 1][...]
            support = jnp.dot(x, w, preferred_element_type=jnp.float32)
            x = _elu(jnp.dot(adj, support, preferred_element_type=jnp.float32) + b)
        w = wb[2 * (nlayers - 1)][...]
        b = wb[2 * (nlayers - 1) + 1][...]
        support = jnp.dot(x, w, preferred_element_type=jnp.float32)
        out = _elu(jnp.dot(adj, support, preferred_element_type=jnp.float32) + b)
        out_ref[0, 0, :] = jnp.max(out, axis=0)
    return body


def kernel(positions, adj, params):
    B, N, _ = positions.shape
    nlayers = len(params)
    latent = params[-1][0].shape[1]

    flat = []
    specs = [
        pl.BlockSpec((1, N, N), lambda i: (i, 0, 0)),
        pl.BlockSpec((1, N, positions.shape[2]), lambda i: (i, 0, 0)),
    ]
    for (w, b) in params:
        flat.append(w)
        specs.append(pl.BlockSpec(w.shape, lambda i: (0, 0)))
        flat.append(b.reshape(1, -1))
        specs.append(pl.BlockSpec((1, b.shape[0]), lambda i: (0, 0)))

    out = pl.pallas_call(
        _encoder_body(nlayers),
        grid=(B,),
        in_specs=specs,
        out_specs=pl.BlockSpec((1, 1, latent), lambda i: (i, 0, 0)),
        out_shape=jax.ShapeDtypeStruct((B, 1, latent), jnp.float32),
        compiler_params=pltpu.CompilerParams(
            dimension_semantics=("arbitrary",),
        ),
    )(adj, positions, *flat)
    return out.reshape(B, latent)


# speedup vs baseline: 1.1863x; 1.0405x over previous
import jax
import jax.numpy as jnp
from jax.experimental import pallas as pl
from jax.experimental.pallas import tpu as pltpu


def _elu(v):
    return jnp.where(v > 0, v, jnp.exp(jnp.minimum(v, 0.0)) - 1.0)


def _pad128(d):
    return -(-d // 128) * 128


def _encoder_body(nlayers):
    def body(*refs):
        adj_ref, pos_ref = refs[0], refs[1]
        out_ref = refs[-1]
        wb = refs[2:-1]
        adj = adj_ref[0]
        x = pos_ref[0]
        for i in range(nlayers):
            w = wb[2 * i][...]
            b = wb[2 * i + 1][...]
            fi, fo = w.shape
            if _pad128(fi) < _pad128(fo):
                y = jnp.dot(jnp.dot(adj, x, preferred_element_type=jnp.float32),
                            w, preferred_element_type=jnp.float32)
            else:
                y = jnp.dot(adj, jnp.dot(x, w, preferred_element_type=jnp.float32),
                            preferred_element_type=jnp.float32)
            x = _elu(y + b)
        out_ref[0, 0, :] = jnp.max(x, axis=0)
    return body


def kernel(positions, adj, params):
    B, N, _ = positions.shape
    nlayers = len(params)
    latent = params[-1][0].shape[1]

    flat = []
    specs = [
        pl.BlockSpec((1, N, N), lambda i: (i, 0, 0)),
        pl.BlockSpec((1, N, positions.shape[2]), lambda i: (i, 0, 0)),
    ]
    for (w, b) in params:
        flat.append(w)
        specs.append(pl.BlockSpec(w.shape, lambda i: (0, 0)))
        flat.append(b.reshape(1, -1))
        specs.append(pl.BlockSpec((1, b.shape[0]), lambda i: (0, 0)))

    out = pl.pallas_call(
        _encoder_body(nlayers),
        grid=(B,),
        in_specs=specs,
        out_specs=pl.BlockSpec((1, 1, latent), lambda i: (i, 0, 0)),
        out_shape=jax.ShapeDtypeStruct((B, 1, latent), jnp.float32),
        compiler_params=pltpu.CompilerParams(
            dimension_semantics=("arbitrary",),
        ),
    )(adj, positions, *flat)
    return out.reshape(B, latent)
